# Initial kernel scaffold; baseline (speedup 1.0000x reference)
import jax
import jax.numpy as jnp
from jax.experimental import pallas as pl


def kernel(x, edge_index, batch, W1, b1, W2, b2, W3, b3, W4, b4, W5, b5, Wl, bl):
    raise NotImplementedError("write your pallas kernel here")


# speedup vs baseline: 6.7279x; 6.7279x over previous
import jax
import jax.numpy as jnp
from jax import lax
from jax.experimental import pallas as pl
from jax.experimental.pallas import tpu as pltpu
from jax.experimental.pallas import tpu_sc as plsc

_N = 10000
_E = 320000
_F = 128
_G = 64
_C = 10

_NC, _NS = 2, 16
_NW = _NC * _NS
_CHUNK = 128
_CPT = 80
_HCPT = _CPT // 2
_EPAD = _NW * _CPT * _CHUNK
_DUMMY = _N
_NPAD = 10240
_RPT = _NPAD // _NS
_BLK = 1024
_NBLK = _NPAD // _BLK

_f32 = jnp.float32


def _sc_mesh():
    return plsc.VectorSubcoreMesh(
        core_axis_name="c", subcore_axis_name="s",
        num_cores=_NC, num_subcores=_NS)


def _sc_deg_body(dsts, ones_h, zeros_h, out, dst_v, ones_v, acc):
    cid = lax.axis_index("c")
    sid = lax.axis_index("s")
    wid = cid * _NS + sid
    pltpu.sync_copy(zeros_h, acc.at[pl.ds(sid * _RPT, _RPT)])
    pltpu.sync_copy(dsts.at[wid], dst_v)
    pltpu.sync_copy(ones_h, ones_v)
    plsc.subcore_barrier()

    def step(k, carry):
        pltpu.sync_copy(ones_v, acc.at[dst_v.at[k]], add=True)
        return carry

    lax.fori_loop(0, _CPT, step, 0)
    plsc.subcore_barrier()
    pltpu.sync_copy(acc.at[pl.ds(sid * _RPT, _RPT)],
                    out.at[cid, pl.ds(sid * _RPT, _RPT)])


def _sc_scatter_body(hs, srcs, dsts, zeros_h, out,
                     src_v, dst_v, rows_a, rows_b, sem_a, sem_b, acc):
    cid = lax.axis_index("c")
    sid = lax.axis_index("s")
    wid = cid * _NS + sid
    pltpu.sync_copy(zeros_h, acc.at[pl.ds(sid * _RPT, _RPT)])
    plsc.subcore_barrier()

    def step(k, carry):
        j0 = 2 * k
        j1 = j0 + 1
        pltpu.make_async_copy(hs.at[src_v.at[j0]], rows_a, sem_a).wait()
        pltpu.async_copy(hs.at[src_v.at[j1]], rows_b, sem_b)
        pltpu.sync_copy(rows_a, acc.at[dst_v.at[j0]], add=True)
        pltpu.make_async_copy(hs.at[src_v.at[j1]], rows_b, sem_b).wait()

        @pl.when(j1 + 1 < _HCPT)
        def _refill():
            pltpu.async_copy(hs.at[src_v.at[j1 + 1]], rows_a, sem_a)

        pltpu.sync_copy(rows_b, acc.at[dst_v.at[j1]], add=True)
        return carry

    for h in range(_CPT // _HCPT):
        pltpu.sync_copy(srcs.at[wid, pl.ds(h * _HCPT, _HCPT)], src_v)
        pltpu.sync_copy(dsts.at[wid, pl.ds(h * _HCPT, _HCPT)], dst_v)
        pltpu.async_copy(hs.at[src_v.at[0]], rows_a, sem_a)
        lax.fori_loop(0, _HCPT // 2, step, 0)
    plsc.subcore_barrier()
    pltpu.sync_copy(acc.at[pl.ds(sid * _RPT, _RPT)],
                    out.at[cid, pl.ds(sid * _RPT, _RPT)])


def _sc_deg(dst3, ones16, zeros16):
    return pl.kernel(
        _sc_deg_body,
        out_type=jax.ShapeDtypeStruct((_NC, _NPAD, 16), _f32),
        mesh=_sc_mesh(),
        scratch_types=[
            pltpu.VMEM((_CPT, _CHUNK), jnp.int32),
            pltpu.VMEM((_CHUNK, 16), _f32),
            pltpu.VMEM_SHARED((_NPAD, 16), _f32),
        ],
    )(dst3, ones16, zeros16)


def _sc_scatter(hs, src3, dst3, zrows):
    return pl.kernel(
        _sc_scatter_body,
        out_type=jax.ShapeDtypeStruct((_NC, _NPAD, _F), _f32),
        mesh=_sc_mesh(),
        scratch_types=[
            pltpu.VMEM((_HCPT, _CHUNK), jnp.int32),
            pltpu.VMEM((_HCPT, _CHUNK), jnp.int32),
            pltpu.VMEM((_CHUNK, _F), _f32),
            pltpu.VMEM((_CHUNK, _F), _f32),
            pltpu.SemaphoreType.DMA,
            pltpu.SemaphoreType.DMA,
            pltpu.VMEM_SHARED((_NPAD, _F), _f32),
        ],
    )(hs, src3, dst3, zrows)


def _tc_first_body(x_ref, w_ref, deg_ref, hs_ref, dv_ref):
    deg = deg_ref[...]
    dv = lax.rsqrt(deg[0] + deg[1] + 1.0)
    dv_ref[...] = dv
    hs_ref[...] = jnp.dot(x_ref[...], w_ref[...],
                          preferred_element_type=_f32) * dv[:, 0:1]


def _tc_first(xp, W1, deg2):
    return pl.pallas_call(
        _tc_first_body,
        grid=(_NBLK,),
        in_specs=[
            pl.BlockSpec((_BLK, _F), lambda i: (i, 0)),
            pl.BlockSpec((_F, _F), lambda i: (0, 0)),
            pl.BlockSpec((_NC, _BLK, 16), lambda i: (0, i, 0)),
        ],
        out_specs=[
            pl.BlockSpec((_BLK, _F), lambda i: (i, 0)),
            pl.BlockSpec((_BLK, 16), lambda i: (i, 0)),
        ],
        out_shape=[
            jax.ShapeDtypeStruct((_NPAD, _F), _f32),
            jax.ShapeDtypeStruct((_NPAD, 16), _f32),
        ],
    )(xp, W1, deg2)


def _tc_mid_body(acc_ref, hs_ref, dv_ref, b_ref, w_ref, out_ref):
    acc = acc_ref[...]
    dv = dv_ref[:, 0:1]
    h = (acc[0] + acc[1] + hs_ref[...]) * dv + b_ref[...]
    out_ref[...] = jnp.dot(h, w_ref[...], preferred_element_type=_f32) * dv


def _tc_mid(acc2, hs, dv, b_prev, W_next):
    return pl.pallas_call(
        _tc_mid_body,
        grid=(_NBLK,),
        in_specs=[
            pl.BlockSpec((_NC, _BLK, _F), lambda i: (0, i, 0)),
            pl.BlockSpec((_BLK, _F), lambda i: (i, 0)),
            pl.BlockSpec((_BLK, 16), lambda i: (i, 0)),
            pl.BlockSpec((1, _F), lambda i: (0, 0)),
            pl.BlockSpec((_F, _F), lambda i: (0, 0)),
        ],
        out_specs=pl.BlockSpec((_BLK, _F), lambda i: (i, 0)),
        out_shape=jax.ShapeDtypeStruct((_NPAD, _F), _f32),
    )(acc2, hs, dv, b_prev, W_next)


def _tc_final_body(acc_ref, hs_ref, dv_ref, b_ref, bat_ref, wl_ref, bl_ref,
                   out_ref, pooled, cnt):
    i = pl.program_id(0)

    @pl.when(i == 0)
    def _init():
        pooled[...] = jnp.zeros_like(pooled)
        cnt[...] = jnp.zeros_like(cnt)

    acc = acc_ref[...]
    dv = dv_ref[:, 0:1]
    h = (acc[0] + acc[1] + hs_ref[...]) * dv + b_ref[...]
    bb = bat_ref[...]
    cls = lax.broadcasted_iota(jnp.int32, (_BLK, _G), 1)
    m = (bb == cls).astype(_f32)
    dn = (((0,), (0,)), ((), ()))
    pooled[...] += lax.dot_general(m, h, dn, preferred_element_type=_f32)
    cnt[...] += lax.dot_general(m, jnp.ones((_BLK, _F), _f32), dn,
                                preferred_element_type=_f32)

    @pl.when(i == pl.num_programs(0) - 1)
    def _finish():
        pm = pooled[...] / jnp.maximum(cnt[...], 1.0)
        out_ref[...] = jnp.dot(pm, wl_ref[...],
                               preferred_element_type=_f32) + bl_ref[...]


def _tc_final(acc2, hs, dv, b5, batp, Wl, bl):
    return pl.pallas_call(
        _tc_final_body,
        grid=(_NBLK,),
        in_specs=[
            pl.BlockSpec((_NC, _BLK, _F), lambda i: (0, i, 0)),
            pl.BlockSpec((_BLK, _F), lambda i: (i, 0)),
            pl.BlockSpec((_BLK, 16), lambda i: (i, 0)),
            pl.BlockSpec((1, _F), lambda i: (0, 0)),
            pl.BlockSpec((_BLK, 1), lambda i: (i, 0)),
            pl.BlockSpec((_F, _C), lambda i: (0, 0)),
            pl.BlockSpec((1, _C), lambda i: (0, 0)),
        ],
        out_specs=pl.BlockSpec((_G, _C), lambda i: (0, 0)),
        out_shape=jax.ShapeDtypeStruct((_G, _C), _f32),
        scratch_shapes=[
            pltpu.VMEM((_G, _F), _f32),
            pltpu.VMEM((_G, _F), _f32),
        ],
    )(acc2, hs, dv, b5, batp, Wl, bl)


def kernel(x, edge_index, batch, W1, b1, W2, b2, W3, b3, W4, b4, W5, b5,
           Wl, bl):
    pad_e = _EPAD - _E
    fill = jnp.full((pad_e,), _DUMMY, jnp.int32)
    src3 = jnp.concatenate([edge_index[0], fill]).reshape(_NW, _CPT, _CHUNK)
    dst3 = jnp.concatenate([edge_index[1], fill]).reshape(_NW, _CPT, _CHUNK)
    xp = jnp.zeros((_NPAD, _F), _f32).at[:_N].set(x)
    batp = jnp.full((_NPAD, 1), _G, jnp.int32).at[:_N, 0].set(batch)
    zrows = jnp.zeros((_RPT, _F), _f32)
    zeros16 = jnp.zeros((_RPT, 16), _f32)
    ones16 = jnp.ones((_CHUNK, 16), _f32)

    deg2 = _sc_deg(dst3, ones16, zeros16)
    hs, dv = _tc_first(xp, W1, deg2)
    for b_prev, W_next in ((b1, W2), (b2, W3), (b3, W4), (b4, W5)):
        acc2 = _sc_scatter(hs, src3, dst3, zrows)
        hs = _tc_mid(acc2, hs, dv, b_prev.reshape(1, _F), W_next)
    acc2 = _sc_scatter(hs, src3, dst3, zrows)
    return _tc_final(acc2, hs, dv, b5.reshape(1, _F), batp, Wl,
                     bl.reshape(1, _C))


# speedup vs baseline: 18.6334x; 2.7696x over previous
import jax
import jax.numpy as jnp
from jax import lax
from jax.experimental import pallas as pl
from jax.experimental.pallas import tpu as pltpu
from jax.experimental.pallas import tpu_sc as plsc

_N = 10000
_E = 320000
_F = 128
_G = 64
_C = 10

_NC, _NS = 2, 16
_NW = _NC * _NS
_CHUNK = 128
_CPT = 80
_HCPT = _CPT // 2
_EPW = _CPT * _CHUNK
_EREAL = _E // _NW
_EDUM = _EPW - _EREAL
_NPAD = 10240
_RPT = _NPAD // _NS
_BLK = 1024
_NBLK = _NPAD // _BLK

_f32 = jnp.float32


def _sc_mesh():
    return plsc.VectorSubcoreMesh(
        core_axis_name="c", subcore_axis_name="s",
        num_cores=_NC, num_subcores=_NS)


def _sc_deg_body(dsts, ones_h, zeros_h, out, dst_v, ones_v, acc):
    cid = lax.axis_index("c")
    sid = lax.axis_index("s")
    wid = cid * _NS + sid
    pltpu.sync_copy(zeros_h, acc.at[pl.ds(sid * _RPT, _RPT)])
    pltpu.sync_copy(dsts.at[wid], dst_v)
    pltpu.sync_copy(ones_h, ones_v)
    plsc.subcore_barrier()

    def step(k, carry):
        pltpu.sync_copy(ones_v, acc.at[dst_v.at[k]], add=True)
        return carry

    lax.fori_loop(0, _CPT, step, 0)
    plsc.subcore_barrier()
    pltpu.sync_copy(acc.at[pl.ds(sid * _RPT, _RPT)],
                    out.at[cid, pl.ds(sid * _RPT, _RPT)])


def _sc_scatter_body(hs, srcs, dsts, zeros_h, out,
                     src_v, dst_v, rows_a, rows_b,
                     sem_ga, sem_gb, sem_sa, sem_sb, acc):
    cid = lax.axis_index("c")
    sid = lax.axis_index("s")
    wid = cid * _NS + sid
    pltpu.sync_copy(zeros_h, acc.at[pl.ds(sid * _RPT, _RPT)])
    plsc.subcore_barrier()

    def step(k, carry):
        j0 = 2 * k
        j1 = j0 + 1
        pltpu.make_async_copy(hs.at[src_v.at[j0]], rows_a, sem_ga).wait()
        pltpu.async_copy(rows_a, acc.at[dst_v.at[j0]], sem_sa, add=True)
        pltpu.make_async_copy(hs.at[src_v.at[j1]], rows_b, sem_gb).wait()
        pltpu.async_copy(rows_b, acc.at[dst_v.at[j1]], sem_sb, add=True)
        pltpu.make_async_copy(rows_a, acc.at[dst_v.at[j0]], sem_sa).wait()

        @pl.when(j1 + 1 < _HCPT)
        def _refill_a():
            pltpu.async_copy(hs.at[src_v.at[j1 + 1]], rows_a, sem_ga)

        pltpu.make_async_copy(rows_b, acc.at[dst_v.at[j1]], sem_sb).wait()

        @pl.when(j1 + 2 < _HCPT)
        def _refill_b():
            pltpu.async_copy(hs.at[src_v.at[j1 + 2]], rows_b, sem_gb)

        return carry

    for h in range(_CPT // _HCPT):
        pltpu.sync_copy(srcs.at[wid, pl.ds(h * _HCPT, _HCPT)], src_v)
        pltpu.sync_copy(dsts.at[wid, pl.ds(h * _HCPT, _HCPT)], dst_v)
        pltpu.async_copy(hs.at[src_v.at[0]], rows_a, sem_ga)
        pltpu.async_copy(hs.at[src_v.at[1]], rows_b, sem_gb)
        lax.fori_loop(0, _HCPT // 2, step, 0)
    plsc.subcore_barrier()
    pltpu.sync_copy(acc.at[pl.ds(sid * _RPT, _RPT)],
                    out.at[cid, pl.ds(sid * _RPT, _RPT)])


def _sc_deg(dst3, ones16, zeros16):
    return pl.kernel(
        _sc_deg_body,
        out_type=jax.ShapeDtypeStruct((_NC, _NPAD, 16), _f32),
        mesh=_sc_mesh(),
        scratch_types=[
            pltpu.VMEM((_CPT, _CHUNK), jnp.int32),
            pltpu.VMEM((_CHUNK, 16), _f32),
            pltpu.VMEM_SHARED((_NPAD, 16), _f32),
        ],
    )(dst3, ones16, zeros16)


def _sc_scatter(hs, src3, dst3, zrows):
    return pl.kernel(
        _sc_scatter_body,
        out_type=jax.ShapeDtypeStruct((_NC, _NPAD, _F), _f32),
        mesh=_sc_mesh(),
        scratch_types=[
            pltpu.VMEM((_HCPT, _CHUNK), jnp.int32),
            pltpu.VMEM((_HCPT, _CHUNK), jnp.int32),
            pltpu.VMEM((_CHUNK, _F), _f32),
            pltpu.VMEM((_CHUNK, _F), _f32),
            pltpu.SemaphoreType.DMA,
            pltpu.SemaphoreType.DMA,
            pltpu.SemaphoreType.DMA,
            pltpu.SemaphoreType.DMA,
            pltpu.VMEM_SHARED((_NPAD, _F), _f32),
        ],
    )(hs, src3, dst3, zrows)


def _tc_first_body(x_ref, w_ref, deg_ref, hs_ref, dv_ref):
    deg = deg_ref[...]
    dv = lax.rsqrt(deg[0] + deg[1] + 1.0)
    dv_ref[...] = dv
    hs_ref[...] = jnp.dot(x_ref[...], w_ref[...],
                          preferred_element_type=_f32) * dv[:, 0:1]


def _tc_first(xp, W1, deg2):
    return pl.pallas_call(
        _tc_first_body,
        grid=(_NBLK,),
        in_specs=[
            pl.BlockSpec((_BLK, _F), lambda i: (i, 0)),
            pl.BlockSpec((_F, _F), lambda i: (0, 0)),
            pl.BlockSpec((_NC, _BLK, 16), lambda i: (0, i, 0)),
        ],
        out_specs=[
            pl.BlockSpec((_BLK, _F), lambda i: (i, 0)),
            pl.BlockSpec((_BLK, 16), lambda i: (i, 0)),
        ],
        out_shape=[
            jax.ShapeDtypeStruct((_NPAD, _F), _f32),
            jax.ShapeDtypeStruct((_NPAD, 16), _f32),
        ],
    )(xp, W1, deg2)


def _tc_mid_body(acc_ref, hs_ref, dv_ref, b_ref, w_ref, out_ref):
    acc = acc_ref[...]
    dv = dv_ref[:, 0:1]
    h = (acc[0] + acc[1] + hs_ref[...]) * dv + b_ref[...]
    out_ref[...] = jnp.dot(h, w_ref[...], preferred_element_type=_f32) * dv


def _tc_mid(acc2, hs, dv, b_prev, W_next):
    return pl.pallas_call(
        _tc_mid_body,
        grid=(_NBLK,),
        in_specs=[
            pl.BlockSpec((_NC, _BLK, _F), lambda i: (0, i, 0)),
            pl.BlockSpec((_BLK, _F), lambda i: (i, 0)),
            pl.BlockSpec((_BLK, 16), lambda i: (i, 0)),
            pl.BlockSpec((1, _F), lambda i: (0, 0)),
            pl.BlockSpec((_F, _F), lambda i: (0, 0)),
        ],
        out_specs=pl.BlockSpec((_BLK, _F), lambda i: (i, 0)),
        out_shape=jax.ShapeDtypeStruct((_NPAD, _F), _f32),
    )(acc2, hs, dv, b_prev, W_next)


def _tc_final_body(acc_ref, hs_ref, dv_ref, b_ref, bat_ref, wl_ref, bl_ref,
                   out_ref, pooled, cnt):
    i = pl.program_id(0)

    @pl.when(i == 0)
    def _init():
        pooled[...] = jnp.zeros_like(pooled)
        cnt[...] = jnp.zeros_like(cnt)

    acc = acc_ref[...]
    dv = dv_ref[:, 0:1]
    h = (acc[0] + acc[1] + hs_ref[...]) * dv + b_ref[...]
    bb = bat_ref[...]
    cls = lax.broadcasted_iota(jnp.int32, (_BLK, _G), 1)
    m = (bb == cls).astype(_f32)
    dn = (((0,), (0,)), ((), ()))
    pooled[...] += lax.dot_general(m, h, dn, preferred_element_type=_f32)
    cnt[...] += lax.dot_general(m, jnp.ones((_BLK, _F), _f32), dn,
                                preferred_element_type=_f32)

    @pl.when(i == pl.num_programs(0) - 1)
    def _finish():
        pm = pooled[...] / jnp.maximum(cnt[...], 1.0)
        out_ref[...] = jnp.dot(pm, wl_ref[...],
                               preferred_element_type=_f32) + bl_ref[...]


def _tc_final(acc2, hs, dv, b5, batp, Wl, bl):
    return pl.pallas_call(
        _tc_final_body,
        grid=(_NBLK,),
        in_specs=[
            pl.BlockSpec((_NC, _BLK, _F), lambda i: (0, i, 0)),
            pl.BlockSpec((_BLK, _F), lambda i: (i, 0)),
            pl.BlockSpec((_BLK, 16), lambda i: (i, 0)),
            pl.BlockSpec((1, _F), lambda i: (0, 0)),
            pl.BlockSpec((_BLK, 1), lambda i: (i, 0)),
            pl.BlockSpec((_F, _C), lambda i: (0, 0)),
            pl.BlockSpec((1, _C), lambda i: (0, 0)),
        ],
        out_specs=pl.BlockSpec((_G, _C), lambda i: (0, 0)),
        out_shape=jax.ShapeDtypeStruct((_G, _C), _f32),
        scratch_shapes=[
            pltpu.VMEM((_G, _F), _f32),
            pltpu.VMEM((_G, _F), _f32),
        ],
    )(acc2, hs, dv, b5, batp, Wl, bl)


def kernel(x, edge_index, batch, W1, b1, W2, b2, W3, b3, W4, b4, W5, b5,
           Wl, bl):
    fill = jnp.broadcast_to(_N + jnp.arange(_EDUM, dtype=jnp.int32),
                            (_NW, _EDUM))
    src3 = jnp.concatenate(
        [edge_index[0].reshape(_NW, _EREAL), fill], axis=1
    ).reshape(_NW, _CPT, _CHUNK)
    dst3 = jnp.concatenate(
        [edge_index[1].reshape(_NW, _EREAL), fill], axis=1
    ).reshape(_NW, _CPT, _CHUNK)
    xp = jnp.zeros((_NPAD, _F), _f32).at[:_N].set(x)
    batp = jnp.full((_NPAD, 1), _G, jnp.int32).at[:_N, 0].set(batch)
    zrows = jnp.zeros((_RPT, _F), _f32)
    zeros16 = jnp.zeros((_RPT, 16), _f32)
    ones16 = jnp.ones((_CHUNK, 16), _f32)

    deg2 = _sc_deg(dst3, ones16, zeros16)
    hs, dv = _tc_first(xp, W1, deg2)
    for b_prev, W_next in ((b1, W2), (b2, W3), (b3, W4), (b4, W5)):
        acc2 = _sc_scatter(hs, src3, dst3, zrows)
        hs = _tc_mid(acc2, hs, dv, b_prev.reshape(1, _F), W_next)
    acc2 = _sc_scatter(hs, src3, dst3, zrows)
    return _tc_final(acc2, hs, dv, b5.reshape(1, _F), batp, Wl,
                     bl.reshape(1, _C))


# speedup vs baseline: 22.1149x; 1.1868x over previous
import jax
import jax.numpy as jnp
from jax import lax
from jax.experimental import pallas as pl
from jax.experimental.pallas import tpu as pltpu
from jax.experimental.pallas import tpu_sc as plsc

_N = 10000
_E = 320000
_F = 128
_G = 64
_C = 10

_NC, _NS = 2, 16
_NW = _NC * _NS
_CHUNK = 128
_CPT = 80
_HCPT = _CPT // 2
_EPW = _CPT * _CHUNK
_EREAL = _E // _NW
_EDUM = _EPW - _EREAL
_NPAD = 10240
_RPT = _NPAD // _NS
_BLK = 1024
_NBLK = _NPAD // _BLK

_f32 = jnp.float32


def _sc_mesh():
    return plsc.VectorSubcoreMesh(
        core_axis_name="c", subcore_axis_name="s",
        num_cores=_NC, num_subcores=_NS)


def _sc_deg_body(dsts, ones_h, zeros_h, out, dst_v, ones_v, acc):
    cid = lax.axis_index("c")
    sid = lax.axis_index("s")
    wid = cid * _NS + sid
    pltpu.sync_copy(zeros_h, acc.at[pl.ds(sid * _RPT, _RPT)])
    pltpu.sync_copy(dsts.at[wid], dst_v)
    pltpu.sync_copy(ones_h, ones_v)
    plsc.subcore_barrier()

    def step(k, carry):
        pltpu.sync_copy(ones_v, acc.at[dst_v.at[k]], add=True)
        return carry

    lax.fori_loop(0, _CPT, step, 0)
    plsc.subcore_barrier()
    pltpu.sync_copy(acc.at[pl.ds(sid * _RPT, _RPT)],
                    out.at[cid, pl.ds(sid * _RPT, _RPT)])


def _sc_scatter_body(hs, srcs, dsts, zeros_h, out,
                     src_v, dst_v, rows_a, rows_b, sem_ga, sem_gb, acc):
    cid = lax.axis_index("c")
    sid = lax.axis_index("s")
    wid = cid * _NS + sid
    pltpu.sync_copy(zeros_h, acc.at[pl.ds(sid * _RPT, _RPT)])
    plsc.subcore_barrier()

    def step(k, carry):
        j0 = 2 * k
        j1 = j0 + 1
        pltpu.make_async_copy(hs.at[src_v.at[j0]], rows_a, sem_ga).wait()
        pltpu.sync_copy(rows_a, acc.at[dst_v.at[j0]], add=True)

        @pl.when(j1 + 1 < _HCPT)
        def _refill_a():
            pltpu.async_copy(hs.at[src_v.at[j1 + 1]], rows_a, sem_ga)

        pltpu.make_async_copy(hs.at[src_v.at[j1]], rows_b, sem_gb).wait()
        pltpu.sync_copy(rows_b, acc.at[dst_v.at[j1]], add=True)

        @pl.when(j1 + 2 < _HCPT)
        def _refill_b():
            pltpu.async_copy(hs.at[src_v.at[j1 + 2]], rows_b, sem_gb)

        return carry

    for h in range(_CPT // _HCPT):
        pltpu.sync_copy(srcs.at[wid, pl.ds(h * _HCPT, _HCPT)], src_v)
        pltpu.sync_copy(dsts.at[wid, pl.ds(h * _HCPT, _HCPT)], dst_v)
        pltpu.async_copy(hs.at[src_v.at[0]], rows_a, sem_ga)
        pltpu.async_copy(hs.at[src_v.at[1]], rows_b, sem_gb)
        lax.fori_loop(0, _HCPT // 2, step, 0)
    plsc.subcore_barrier()
    pltpu.sync_copy(acc.at[pl.ds(sid * _RPT, _RPT)],
                    out.at[cid, pl.ds(sid * _RPT, _RPT)])


def _sc_deg(dst3, ones_rows, zrows):
    return pl.kernel(
        _sc_deg_body,
        out_type=jax.ShapeDtypeStruct((_NC, _NPAD, _F), _f32),
        mesh=_sc_mesh(),
        scratch_types=[
            pltpu.VMEM((_CPT, _CHUNK), jnp.int32),
            pltpu.VMEM((_CHUNK, _F), _f32),
            pltpu.VMEM_SHARED((_NPAD, _F), _f32),
        ],
    )(dst3, ones_rows, zrows)


def _sc_scatter(hs, src3, dst3, zrows):
    return pl.kernel(
        _sc_scatter_body,
        out_type=jax.ShapeDtypeStruct((_NC, _NPAD, _F), _f32),
        mesh=_sc_mesh(),
        scratch_types=[
            pltpu.VMEM((_HCPT, _CHUNK), jnp.int32),
            pltpu.VMEM((_HCPT, _CHUNK), jnp.int32),
            pltpu.VMEM((_CHUNK, _F), _f32),
            pltpu.VMEM((_CHUNK, _F), _f32),
            pltpu.SemaphoreType.DMA,
            pltpu.SemaphoreType.DMA,
            pltpu.VMEM_SHARED((_NPAD, _F), _f32),
        ],
    )(hs, src3, dst3, zrows)


def _tc_first_body(x_ref, w_ref, deg_ref, hs_ref, dv_ref):
    deg = deg_ref[...]
    dv = lax.rsqrt(deg[0] + deg[1] + 1.0)
    dv_ref[...] = dv[:, :16]
    hs_ref[...] = jnp.dot(x_ref[...], w_ref[...],
                          preferred_element_type=_f32) * dv[:, 0:1]


def _tc_first(xp, W1, deg2):
    return pl.pallas_call(
        _tc_first_body,
        grid=(_NBLK,),
        in_specs=[
            pl.BlockSpec((_BLK, _F), lambda i: (i, 0)),
            pl.BlockSpec((_F, _F), lambda i: (0, 0)),
            pl.BlockSpec((_NC, _BLK, _F), lambda i: (0, i, 0)),
        ],
        out_specs=[
            pl.BlockSpec((_BLK, _F), lambda i: (i, 0)),
            pl.BlockSpec((_BLK, 16), lambda i: (i, 0)),
        ],
        out_shape=[
            jax.ShapeDtypeStruct((_NPAD, _F), _f32),
            jax.ShapeDtypeStruct((_NPAD, 16), _f32),
        ],
    )(xp, W1, deg2)


def _tc_mid_body(acc_ref, hs_ref, dv_ref, b_ref, w_ref, out_ref):
    acc = acc_ref[...]
    dv = dv_ref[:, 0:1]
    h = (acc[0] + acc[1] + hs_ref[...]) * dv + b_ref[...]
    out_ref[...] = jnp.dot(h, w_ref[...], preferred_element_type=_f32) * dv


def _tc_mid(acc2, hs, dv, b_prev, W_next):
    return pl.pallas_call(
        _tc_mid_body,
        grid=(_NBLK,),
        in_specs=[
            pl.BlockSpec((_NC, _BLK, _F), lambda i: (0, i, 0)),
            pl.BlockSpec((_BLK, _F), lambda i: (i, 0)),
            pl.BlockSpec((_BLK, 16), lambda i: (i, 0)),
            pl.BlockSpec((1, _F), lambda i: (0, 0)),
            pl.BlockSpec((_F, _F), lambda i: (0, 0)),
        ],
        out_specs=pl.BlockSpec((_BLK, _F), lambda i: (i, 0)),
        out_shape=jax.ShapeDtypeStruct((_NPAD, _F), _f32),
    )(acc2, hs, dv, b_prev, W_next)


def _tc_final_body(acc_ref, hs_ref, dv_ref, b_ref, bat_ref, wl_ref, bl_ref,
                   out_ref, pooled, cnt):
    i = pl.program_id(0)

    @pl.when(i == 0)
    def _init():
        pooled[...] = jnp.zeros_like(pooled)
        cnt[...] = jnp.zeros_like(cnt)

    acc = acc_ref[...]
    dv = dv_ref[:, 0:1]
    h = (acc[0] + acc[1] + hs_ref[...]) * dv + b_ref[...]
    bb = bat_ref[...]
    cls = lax.broadcasted_iota(jnp.int32, (_BLK, _G), 1)
    m = (bb == cls).astype(_f32)
    dn = (((0,), (0,)), ((), ()))
    pooled[...] += lax.dot_general(m, h, dn, preferred_element_type=_f32)
    cnt[...] += lax.dot_general(m, jnp.ones((_BLK, _F), _f32), dn,
                                preferred_element_type=_f32)

    @pl.when(i == pl.num_programs(0) - 1)
    def _finish():
        pm = pooled[...] / jnp.maximum(cnt[...], 1.0)
        out_ref[...] = jnp.dot(pm, wl_ref[...],
                               preferred_element_type=_f32) + bl_ref[...]


def _tc_final(acc2, hs, dv, b5, batp, Wl, bl):
    return pl.pallas_call(
        _tc_final_body,
        grid=(_NBLK,),
        in_specs=[
            pl.BlockSpec((_NC, _BLK, _F), lambda i: (0, i, 0)),
            pl.BlockSpec((_BLK, _F), lambda i: (i, 0)),
            pl.BlockSpec((_BLK, 16), lambda i: (i, 0)),
            pl.BlockSpec((1, _F), lambda i: (0, 0)),
            pl.BlockSpec((_BLK, 1), lambda i: (i, 0)),
            pl.BlockSpec((_F, _C), lambda i: (0, 0)),
            pl.BlockSpec((1, _C), lambda i: (0, 0)),
        ],
        out_specs=pl.BlockSpec((_G, _C), lambda i: (0, 0)),
        out_shape=jax.ShapeDtypeStruct((_G, _C), _f32),
        scratch_shapes=[
            pltpu.VMEM((_G, _F), _f32),
            pltpu.VMEM((_G, _F), _f32),
        ],
    )(acc2, hs, dv, b5, batp, Wl, bl)


def kernel(x, edge_index, batch, W1, b1, W2, b2, W3, b3, W4, b4, W5, b5,
           Wl, bl):
    fill = jnp.broadcast_to(_N + jnp.arange(_EDUM, dtype=jnp.int32),
                            (_NW, _EDUM))
    src3 = jnp.concatenate(
        [edge_index[0].reshape(_NW, _EREAL), fill], axis=1
    ).reshape(_NW, _CPT, _CHUNK)
    dst3 = jnp.concatenate(
        [edge_index[1].reshape(_NW, _EREAL), fill], axis=1
    ).reshape(_NW, _CPT, _CHUNK)
    xp = jnp.zeros((_NPAD, _F), _f32).at[:_N].set(x)
    batp = jnp.full((_NPAD, 1), _G, jnp.int32).at[:_N, 0].set(batch)
    zrows = jnp.zeros((_RPT, _F), _f32)
    ones_rows = jnp.ones((_CHUNK, _F), _f32)

    deg2 = _sc_deg(dst3, ones_rows, zrows)
    hs, dv = _tc_first(xp, W1, deg2)
    for b_prev, W_next in ((b1, W2), (b2, W3), (b3, W4), (b4, W5)):
        acc2 = _sc_scatter(hs, src3, dst3, zrows)
        hs = _tc_mid(acc2, hs, dv, b_prev.reshape(1, _F), W_next)
    acc2 = _sc_scatter(hs, src3, dst3, zrows)
    return _tc_final(acc2, hs, dv, b5.reshape(1, _F), batp, Wl,
                     bl.reshape(1, _C))
